# baseline (speedup 1.0000x reference)
import functools
import jax
import jax.numpy as jnp
from jax import lax
from jax.experimental import pallas as pl
from jax.experimental.pallas import tpu as pltpu
from jax.experimental.pallas import tpu_sc as plsc

_NC = 2
_NS = 16
_NW = _NC * _NS
_CS = 32
_UNROLL = 64


def _sc_body(batch, seq_len, embed, x_hbm, pos_hbm, out_hbm,
             posbuf, xb0, xb1, ld0, ld1, st0, st1, pld):
    rows_per_w = seq_len // _NW
    n_chunks = rows_per_w // _CS
    n_steps = n_chunks * batch
    gpr = embed // (_UNROLL * 16)
    wid = lax.axis_index("s") * _NC + lax.axis_index("c")
    wbase = wid * rows_per_w
    xbufs = (xb0, xb1)
    lds = (ld0, ld1)
    sts = (st0, st1)

    def start_xload(i):
        c, b = divmod(i, batch)
        return pltpu.async_copy(
            x_hbm.at[b, pl.ds(wbase + c * _CS, _CS)], xbufs[i % 2], lds[i % 2]
        )

    pos_desc = pltpu.async_copy(pos_hbm.at[pl.ds(wbase, _CS)], posbuf, pld)
    x_descs = {0: start_xload(0)}
    st_descs = {}
    for i in range(n_steps):
        c, b = divmod(i, batch)
        k = i % 2
        if i + 1 < n_steps:
            if i >= 1:
                st_descs[i - 1].wait()
            x_descs[i + 1] = start_xload(i + 1)
        if b == 0:
            pos_desc.wait()
        x_descs[i].wait()
        xb = xbufs[k]

        def group_add(g, carry, xb=xb):
            r = g // gpr
            colbase = (g % gpr) * (_UNROLL * 16)
            for p in range(_UNROLL // 8):
                cols = [colbase + (p * 8 + u) * 16 for u in range(8)]
                pv = [posbuf[r, pl.ds(c0, 16)] for c0 in cols]
                for c0, v in zip(cols, pv):
                    plsc.addupdate(xb.at[r, pl.ds(c0, 16)], v)
            return carry

        lax.fori_loop(0, _CS * gpr, group_add, 0)
        if b == batch - 1 and c + 1 < n_chunks:
            pos_desc = pltpu.async_copy(
                pos_hbm.at[pl.ds(wbase + (c + 1) * _CS, _CS)], posbuf, pld
            )
        st_descs[i] = pltpu.async_copy(
            xb, out_hbm.at[b, pl.ds(wbase + c * _CS, _CS)], sts[k]
        )
    st_descs[n_steps - 2].wait()
    st_descs[n_steps - 1].wait()


def kernel(x, pos_table):
    batch, seq_len, embed = x.shape
    mesh = plsc.VectorSubcoreMesh(core_axis_name="c", subcore_axis_name="s")
    run = pl.kernel(
        functools.partial(_sc_body, batch, seq_len, embed),
        out_type=jax.ShapeDtypeStruct((batch, seq_len, embed), x.dtype),
        mesh=mesh,
        scratch_types=[
            pltpu.VMEM((_CS, embed), jnp.float32),
            pltpu.VMEM((_CS, embed), jnp.float32),
            pltpu.VMEM((_CS, embed), jnp.float32),
            pltpu.SemaphoreType.DMA,
            pltpu.SemaphoreType.DMA,
            pltpu.SemaphoreType.DMA,
            pltpu.SemaphoreType.DMA,
            pltpu.SemaphoreType.DMA,
        ],
    )
    return run(x, pos_table)


# speedup vs baseline: 1.0444x; 1.0444x over previous
import functools
import jax
import jax.numpy as jnp
from jax import lax
from jax.experimental import pallas as pl
from jax.experimental.pallas import tpu as pltpu
from jax.experimental.pallas import tpu_sc as plsc

_NC = 2
_NS = 16
_NW = _NC * _NS
_CS = 32
_UNROLL = 32


def _sc_body(batch, seq_len, embed, x_hbm, pos_hbm, out_hbm,
             posbuf, xb0, xb1, ld0, ld1, st0, st1, pld):
    rows_per_w = seq_len // _NW
    n_chunks = rows_per_w // _CS
    n_steps = n_chunks * batch
    gpr = embed // (_UNROLL * 16)
    wid = lax.axis_index("s") * _NC + lax.axis_index("c")
    wbase = wid * rows_per_w
    xbufs = (xb0, xb1)
    lds = (ld0, ld1)
    sts = (st0, st1)

    def start_xload(i):
        c, b = divmod(i, batch)
        return pltpu.async_copy(
            x_hbm.at[b, pl.ds(wbase + c * _CS, _CS)], xbufs[i % 2], lds[i % 2]
        )

    pos_desc = pltpu.async_copy(pos_hbm.at[pl.ds(wbase, _CS)], posbuf, pld)
    x_descs = {0: start_xload(0)}
    st_descs = {}
    for i in range(n_steps):
        c, b = divmod(i, batch)
        k = i % 2
        if i + 1 < n_steps:
            if i >= 1:
                st_descs[i - 1].wait()
            x_descs[i + 1] = start_xload(i + 1)
        if b == 0:
            pos_desc.wait()
        x_descs[i].wait()
        xb = xbufs[k]

        def group_add(g, carry, xb=xb):
            r = g // gpr
            colbase = (g % gpr) * (_UNROLL * 16)
            for p in range(_UNROLL // 8):
                cols = [colbase + (p * 8 + u) * 16 for u in range(8)]
                pv = [posbuf[r, pl.ds(c0, 16)] for c0 in cols]
                for c0, v in zip(cols, pv):
                    plsc.addupdate(xb.at[r, pl.ds(c0, 16)], v)
            return carry

        lax.fori_loop(0, _CS * gpr, group_add, 0)
        if b == batch - 1 and c + 1 < n_chunks:
            pos_desc = pltpu.async_copy(
                pos_hbm.at[pl.ds(wbase + (c + 1) * _CS, _CS)], posbuf, pld
            )
        st_descs[i] = pltpu.async_copy(
            xb, out_hbm.at[b, pl.ds(wbase + c * _CS, _CS)], sts[k]
        )
    st_descs[n_steps - 2].wait()
    st_descs[n_steps - 1].wait()


def kernel(x, pos_table):
    batch, seq_len, embed = x.shape
    mesh = plsc.VectorSubcoreMesh(core_axis_name="c", subcore_axis_name="s")
    run = pl.kernel(
        functools.partial(_sc_body, batch, seq_len, embed),
        out_type=jax.ShapeDtypeStruct((batch, seq_len, embed), x.dtype),
        mesh=mesh,
        scratch_types=[
            pltpu.VMEM((_CS, embed), jnp.float32),
            pltpu.VMEM((_CS, embed), jnp.float32),
            pltpu.VMEM((_CS, embed), jnp.float32),
            pltpu.SemaphoreType.DMA,
            pltpu.SemaphoreType.DMA,
            pltpu.SemaphoreType.DMA,
            pltpu.SemaphoreType.DMA,
            pltpu.SemaphoreType.DMA,
        ],
    )
    return run(x, pos_table)


# speedup vs baseline: 1.0749x; 1.0292x over previous
import functools
import jax
import jax.numpy as jnp
from jax import lax
from jax.experimental import pallas as pl
from jax.experimental.pallas import tpu as pltpu
from jax.experimental.pallas import tpu_sc as plsc

_NC = 2
_NS = 16
_NW = _NC * _NS
_CS = 32
_UNROLL = 16


def _sc_body(batch, seq_len, embed, x_hbm, pos_hbm, out_hbm,
             posbuf, xb0, xb1, ld0, ld1, st0, st1, pld):
    rows_per_w = seq_len // _NW
    n_chunks = rows_per_w // _CS
    n_steps = n_chunks * batch
    gpr = embed // (_UNROLL * 16)
    wid = lax.axis_index("s") * _NC + lax.axis_index("c")
    wbase = wid * rows_per_w
    xbufs = (xb0, xb1)
    lds = (ld0, ld1)
    sts = (st0, st1)

    def start_xload(i):
        c, b = divmod(i, batch)
        return pltpu.async_copy(
            x_hbm.at[b, pl.ds(wbase + c * _CS, _CS)], xbufs[i % 2], lds[i % 2]
        )

    pos_desc = pltpu.async_copy(pos_hbm.at[pl.ds(wbase, _CS)], posbuf, pld)
    x_descs = {0: start_xload(0)}
    st_descs = {}
    for i in range(n_steps):
        c, b = divmod(i, batch)
        k = i % 2
        if i + 1 < n_steps:
            if i >= 1:
                st_descs[i - 1].wait()
            x_descs[i + 1] = start_xload(i + 1)
        if b == 0:
            pos_desc.wait()
        x_descs[i].wait()
        xb = xbufs[k]

        @plsc.parallel_loop(0, _CS * gpr)
        def group_add(g, xb=xb):
            r = g // gpr
            colbase = (g % gpr) * (_UNROLL * 16)
            for p in range(_UNROLL // 8):
                cols = [colbase + (p * 8 + u) * 16 for u in range(8)]
                pv = [posbuf[r, pl.ds(c0, 16)] for c0 in cols]
                for c0, v in zip(cols, pv):
                    plsc.addupdate(xb.at[r, pl.ds(c0, 16)], v)
        if b == batch - 1 and c + 1 < n_chunks:
            pos_desc = pltpu.async_copy(
                pos_hbm.at[pl.ds(wbase + (c + 1) * _CS, _CS)], posbuf, pld
            )
        st_descs[i] = pltpu.async_copy(
            xb, out_hbm.at[b, pl.ds(wbase + c * _CS, _CS)], sts[k]
        )
    st_descs[n_steps - 2].wait()
    st_descs[n_steps - 1].wait()


def kernel(x, pos_table):
    batch, seq_len, embed = x.shape
    mesh = plsc.VectorSubcoreMesh(core_axis_name="c", subcore_axis_name="s")
    run = pl.kernel(
        functools.partial(_sc_body, batch, seq_len, embed),
        out_type=jax.ShapeDtypeStruct((batch, seq_len, embed), x.dtype),
        mesh=mesh,
        scratch_types=[
            pltpu.VMEM((_CS, embed), jnp.float32),
            pltpu.VMEM((_CS, embed), jnp.float32),
            pltpu.VMEM((_CS, embed), jnp.float32),
            pltpu.SemaphoreType.DMA,
            pltpu.SemaphoreType.DMA,
            pltpu.SemaphoreType.DMA,
            pltpu.SemaphoreType.DMA,
            pltpu.SemaphoreType.DMA,
        ],
    )
    return run(x, pos_table)


# speedup vs baseline: 1.1095x; 1.0322x over previous
import functools
import jax
import jax.numpy as jnp
from jax import lax
from jax.experimental import pallas as pl
from jax.experimental.pallas import tpu as pltpu
from jax.experimental.pallas import tpu_sc as plsc

_NC = 2
_NS = 16
_NW = _NC * _NS
_CS = 16
_NBUF = 4
_UNROLL = 16


def _sc_body(batch, seq_len, embed, x_hbm, pos_hbm, out_hbm, posbuf, *rest):
    xbufs = rest[:_NBUF]
    lds = rest[_NBUF:2 * _NBUF]
    sts = rest[2 * _NBUF:3 * _NBUF]
    pld = rest[3 * _NBUF]
    rows_per_w = seq_len // _NW
    n_chunks = rows_per_w // _CS
    n_steps = n_chunks * batch
    gpr = embed // (_UNROLL * 16)
    wid = lax.axis_index("s") * _NC + lax.axis_index("c")
    wbase = wid * rows_per_w

    def start_xload(i):
        c, b = divmod(i, batch)
        return pltpu.async_copy(
            x_hbm.at[b, pl.ds(wbase + c * _CS, _CS)],
            xbufs[i % _NBUF], lds[i % _NBUF],
        )

    pos_desc = pltpu.async_copy(pos_hbm.at[pl.ds(wbase, _CS)], posbuf, pld)
    x_descs = {}
    st_descs = {}
    for i in range(min(_NBUF - 2, n_steps)):
        x_descs[i] = start_xload(i)
    for i in range(n_steps):
        c, b = divmod(i, batch)
        k = i % _NBUF
        j = i + _NBUF - 2
        if j < n_steps:
            if j - _NBUF >= 0:
                st_descs[j - _NBUF].wait()
            x_descs[j] = start_xload(j)
        if b == 0:
            pos_desc.wait()
        x_descs[i].wait()
        xb = xbufs[k]

        @plsc.parallel_loop(0, _CS * gpr)
        def group_add(g, xb=xb):
            r = g // gpr
            colbase = (g % gpr) * (_UNROLL * 16)
            for p in range(_UNROLL // 8):
                cols = [colbase + (p * 8 + u) * 16 for u in range(8)]
                pv = [posbuf[r, pl.ds(c0, 16)] for c0 in cols]
                for c0, v in zip(cols, pv):
                    plsc.addupdate(xb.at[r, pl.ds(c0, 16)], v)

        if b == batch - 1 and c + 1 < n_chunks:
            pos_desc = pltpu.async_copy(
                pos_hbm.at[pl.ds(wbase + (c + 1) * _CS, _CS)], posbuf, pld
            )
        st_descs[i] = pltpu.async_copy(
            xb, out_hbm.at[b, pl.ds(wbase + c * _CS, _CS)], sts[k]
        )
    for i in range(max(0, n_steps - _NBUF), n_steps):
        st_descs[i].wait()


def kernel(x, pos_table):
    batch, seq_len, embed = x.shape
    mesh = plsc.VectorSubcoreMesh(core_axis_name="c", subcore_axis_name="s")
    run = pl.kernel(
        functools.partial(_sc_body, batch, seq_len, embed),
        out_type=jax.ShapeDtypeStruct((batch, seq_len, embed), x.dtype),
        mesh=mesh,
        scratch_types=(
            [pltpu.VMEM((_CS, embed), jnp.float32)]
            + [pltpu.VMEM((_CS, embed), jnp.float32) for _ in range(_NBUF)]
            + [pltpu.SemaphoreType.DMA for _ in range(2 * _NBUF + 1)]
        ),
    )
    return run(x, pos_table)


# speedup vs baseline: 1.1221x; 1.0114x over previous
import functools
import jax
import jax.numpy as jnp
from jax import lax
from jax.experimental import pallas as pl
from jax.experimental.pallas import tpu as pltpu
from jax.experimental.pallas import tpu_sc as plsc

_NC = 2
_NS = 16
_NW = _NC * _NS
_CS = 16
_NBUF = 6
_UNROLL = 16


def _sc_body(batch, seq_len, embed, x_hbm, pos_hbm, out_hbm, posbuf, *rest):
    xbufs = rest[:_NBUF]
    lds = rest[_NBUF:2 * _NBUF]
    sts = rest[2 * _NBUF:3 * _NBUF]
    pld = rest[3 * _NBUF]
    rows_per_w = seq_len // _NW
    n_chunks = rows_per_w // _CS
    n_steps = n_chunks * batch
    gpr = embed // (_UNROLL * 16)
    wid = lax.axis_index("s") * _NC + lax.axis_index("c")
    wbase = wid * rows_per_w

    def start_xload(i):
        c, b = divmod(i, batch)
        return pltpu.async_copy(
            x_hbm.at[b, pl.ds(wbase + c * _CS, _CS)],
            xbufs[i % _NBUF], lds[i % _NBUF],
        )

    pos_desc = pltpu.async_copy(pos_hbm.at[pl.ds(wbase, _CS)], posbuf, pld)
    x_descs = {}
    st_descs = {}
    for i in range(min(_NBUF - 2, n_steps)):
        x_descs[i] = start_xload(i)
    for i in range(n_steps):
        c, b = divmod(i, batch)
        k = i % _NBUF
        j = i + _NBUF - 2
        if j < n_steps:
            if j - _NBUF >= 0:
                st_descs[j - _NBUF].wait()
            x_descs[j] = start_xload(j)
        if b == 0:
            pos_desc.wait()
        x_descs[i].wait()
        xb = xbufs[k]

        @plsc.parallel_loop(0, _CS * gpr)
        def group_add(g, xb=xb):
            r = g // gpr
            colbase = (g % gpr) * (_UNROLL * 16)
            for p in range(_UNROLL // 8):
                cols = [colbase + (p * 8 + u) * 16 for u in range(8)]
                pv = [posbuf[r, pl.ds(c0, 16)] for c0 in cols]
                for c0, v in zip(cols, pv):
                    plsc.addupdate(xb.at[r, pl.ds(c0, 16)], v)

        if b == batch - 1 and c + 1 < n_chunks:
            pos_desc = pltpu.async_copy(
                pos_hbm.at[pl.ds(wbase + (c + 1) * _CS, _CS)], posbuf, pld
            )
        st_descs[i] = pltpu.async_copy(
            xb, out_hbm.at[b, pl.ds(wbase + c * _CS, _CS)], sts[k]
        )
    for i in range(max(0, n_steps - _NBUF), n_steps):
        st_descs[i].wait()


def kernel(x, pos_table):
    batch, seq_len, embed = x.shape
    mesh = plsc.VectorSubcoreMesh(core_axis_name="c", subcore_axis_name="s")
    run = pl.kernel(
        functools.partial(_sc_body, batch, seq_len, embed),
        out_type=jax.ShapeDtypeStruct((batch, seq_len, embed), x.dtype),
        mesh=mesh,
        scratch_types=(
            [pltpu.VMEM((_CS, embed), jnp.float32)]
            + [pltpu.VMEM((_CS, embed), jnp.float32) for _ in range(_NBUF)]
            + [pltpu.SemaphoreType.DMA for _ in range(2 * _NBUF + 1)]
        ),
    )
    return run(x, pos_table)


# speedup vs baseline: 1.2455x; 1.1100x over previous
import functools
import jax
import jax.numpy as jnp
from jax import lax
from jax.experimental import pallas as pl
from jax.experimental.pallas import tpu as pltpu
from jax.experimental.pallas import tpu_sc as plsc

_NC = 2
_NS = 16
_NW = _NC * _NS
_CS = 16
_NBUF = 5
_UNROLL = 16


def _sc_body(batch, seq_len, embed, x_hbm, pos_hbm, out_hbm, *rest):
    posbufs = rest[:2]
    xbufs = rest[2:2 + _NBUF]
    lds = rest[2 + _NBUF:2 + 2 * _NBUF]
    sts = rest[2 + 2 * _NBUF:2 + 3 * _NBUF]
    plds = rest[2 + 3 * _NBUF:2 + 3 * _NBUF + 2]
    rows_per_w = seq_len // _NW
    n_chunks = rows_per_w // _CS
    n_steps = n_chunks * batch
    gpr = embed // (_UNROLL * 16)
    wid = lax.axis_index("s") * _NC + lax.axis_index("c")
    wbase = wid * rows_per_w

    def start_xload(i):
        c, b = divmod(i, batch)
        return pltpu.async_copy(
            x_hbm.at[b, pl.ds(wbase + c * _CS, _CS)],
            xbufs[i % _NBUF], lds[i % _NBUF],
        )

    def start_posload(c):
        return pltpu.async_copy(
            pos_hbm.at[pl.ds(wbase + c * _CS, _CS)], posbufs[c % 2], plds[c % 2]
        )

    pos_descs = {c: start_posload(c) for c in range(min(2, n_chunks))}
    x_descs = {}
    st_descs = {}
    for i in range(min(_NBUF - 2, n_steps)):
        x_descs[i] = start_xload(i)
    for i in range(n_steps):
        c, b = divmod(i, batch)
        k = i % _NBUF
        j = i + _NBUF - 2
        if j < n_steps:
            if j - _NBUF >= 0:
                st_descs[j - _NBUF].wait()
            x_descs[j] = start_xload(j)
        if b == 0:
            pos_descs[c].wait()
        x_descs[i].wait()
        xb = xbufs[k]
        pb = posbufs[c % 2]

        @plsc.parallel_loop(0, _CS * gpr)
        def group_add(g, xb=xb, pb=pb):
            r = g // gpr
            colbase = (g % gpr) * (_UNROLL * 16)
            for p in range(_UNROLL // 8):
                cols = [colbase + (p * 8 + u) * 16 for u in range(8)]
                pv = [pb[r, pl.ds(c0, 16)] for c0 in cols]
                for c0, v in zip(cols, pv):
                    plsc.addupdate(xb.at[r, pl.ds(c0, 16)], v)

        if b == batch - 1 and c + 2 < n_chunks:
            pos_descs[c + 2] = start_posload(c + 2)
        st_descs[i] = pltpu.async_copy(
            xb, out_hbm.at[b, pl.ds(wbase + c * _CS, _CS)], sts[k]
        )
    for i in range(max(0, n_steps - _NBUF), n_steps):
        st_descs[i].wait()


def kernel(x, pos_table):
    batch, seq_len, embed = x.shape
    mesh = plsc.VectorSubcoreMesh(core_axis_name="c", subcore_axis_name="s")
    run = pl.kernel(
        functools.partial(_sc_body, batch, seq_len, embed),
        out_type=jax.ShapeDtypeStruct((batch, seq_len, embed), x.dtype),
        mesh=mesh,
        scratch_types=(
            [pltpu.VMEM((_CS, embed), jnp.float32) for _ in range(2)]
            + [pltpu.VMEM((_CS, embed), jnp.float32) for _ in range(_NBUF)]
            + [pltpu.SemaphoreType.DMA for _ in range(2 * _NBUF + 2)]
        ),
    )
    return run(x, pos_table)
